# baseline (speedup 1.0000x reference)
import jax
import jax.numpy as jnp
from jax import lax
from jax.experimental import pallas as pl
from jax.experimental.pallas import tpu as pltpu
from jax.experimental.pallas import tpu_sc as plsc

N = 10000
E = 320000
D = 128
H = 64

NC = 2
NS = 16
NW = NC * NS
CH = 128
NCHUNK = 80
EPAD = NW * NCHUNK * CH
NP = 10240
RPT = NP // NS
KD = 8

_sc_mesh = plsc.VectorSubcoreMesh(
    core_axis_name="c", subcore_axis_name="s", num_cores=NC, num_subcores=NS)


def _sc_deg_body(dst_hbm, ones_hbm, zeros_hbm, out_hbm, dstv, onesv, acc, sem):
    c = lax.axis_index("c")
    s = lax.axis_index("s")
    pltpu.sync_copy(dst_hbm.at[c, s], dstv)
    pltpu.sync_copy(ones_hbm, onesv)
    pltpu.sync_copy(zeros_hbm.at[pl.ds(s * RPT, RPT)],
                    acc.at[pl.ds(s * RPT, RPT)])
    plsc.subcore_barrier()

    def step(j, carry):
        pltpu.sync_copy(onesv, acc.at[dstv.at[j]], add=True)
        return carry

    lax.fori_loop(0, NCHUNK, step, 0)
    plsc.subcore_barrier()
    pltpu.sync_copy(acc.at[pl.ds(s * RPT, RPT)],
                    out_hbm.at[c, pl.ds(s * RPT, RPT)])


_sc_deg = pl.kernel(
    _sc_deg_body,
    out_type=jax.ShapeDtypeStruct((NC, NP, 16), jnp.float32),
    mesh=_sc_mesh,
    compiler_params=pltpu.CompilerParams(use_tc_tiling_on_sc=False),
    scratch_types=[
        pltpu.VMEM((NCHUNK, CH), jnp.int32),
        pltpu.VMEM((CH, 16), jnp.float32),
        pltpu.VMEM_SHARED((NP, 16), jnp.float32),
        pltpu.SemaphoreType.DMA,
    ],
)


def _sc_agg_body(g_hbm, src_hbm, dst_hbm, zeros_hbm, out_hbm,
                 srcv, dstv, rows, acc, sem):
    c = lax.axis_index("c")
    s = lax.axis_index("s")
    pltpu.sync_copy(src_hbm.at[c, s], srcv)
    pltpu.sync_copy(dst_hbm.at[c, s], dstv)
    pltpu.sync_copy(zeros_hbm.at[pl.ds(s * RPT, RPT)],
                    acc.at[pl.ds(s * RPT, RPT)])
    plsc.subcore_barrier()

    def step(j, carry):
        pltpu.async_copy(g_hbm.at[srcv.at[j]], rows.at[0], sem).wait()
        pltpu.sync_copy(rows.at[0], acc.at[dstv.at[j]], add=True)
        return carry

    lax.fori_loop(0, NCHUNK, step, 0)
    plsc.subcore_barrier()
    pltpu.sync_copy(acc.at[pl.ds(s * RPT, RPT)],
                    out_hbm.at[c, pl.ds(s * RPT, RPT)])


_sc_agg = pl.kernel(
    _sc_agg_body,
    out_type=jax.ShapeDtypeStruct((NC, NP, H), jnp.float32),
    mesh=_sc_mesh,
    compiler_params=pltpu.CompilerParams(use_tc_tiling_on_sc=False),
    scratch_types=[
        pltpu.VMEM((NCHUNK, CH), jnp.int32),
        pltpu.VMEM((NCHUNK, CH), jnp.int32),
        pltpu.VMEM((KD, CH, H), jnp.float32),
        pltpu.VMEM_SHARED((NP, H), jnp.float32),
        pltpu.SemaphoreType.DMA,
    ],
)


BLK = 2000


def _tc1_body(deg_ref, x_ref, w1_ref, dinv_ref, g1_ref):
    deg = deg_ref[:, 0] + deg_ref[:, 1] + 1.0
    dinv = lax.rsqrt(deg)[:, None]
    h = lax.dot_general(x_ref[...], w1_ref[...], (((1,), (1,)), ((), ())),
                        preferred_element_type=jnp.float32)
    dinv_ref[...] = dinv
    g1_ref[...] = h * dinv


def _tc1(deg2, x, w1):
    return pl.pallas_call(
        _tc1_body,
        grid=(N // BLK,),
        in_specs=[
            pl.BlockSpec((BLK, 2), lambda i: (i, 0)),
            pl.BlockSpec((BLK, D), lambda i: (i, 0)),
            pl.BlockSpec((H, D), lambda i: (0, 0)),
        ],
        out_specs=[
            pl.BlockSpec((BLK, 1), lambda i: (i, 0)),
            pl.BlockSpec((BLK, H), lambda i: (i, 0)),
        ],
        out_shape=[
            jax.ShapeDtypeStruct((N, 1), jnp.float32),
            jax.ShapeDtypeStruct((N, H), jnp.float32),
        ],
    )(deg2, x, w1)


def _tc2_body(aggp_ref, g1_ref, dinv_ref, b1_ref, w2_ref, g2_ref):
    agg = aggp_ref[0] + aggp_ref[1] + g1_ref[...]
    h = jnp.maximum(agg * dinv_ref[...] + b1_ref[...], 0.0)
    g2 = lax.dot_general(h, w2_ref[...], (((1,), (1,)), ((), ())),
                         preferred_element_type=jnp.float32)
    g2_ref[...] = g2 * dinv_ref[...]


def _tc2(aggp, g1, dinv, b1, w2):
    return pl.pallas_call(
        _tc2_body,
        grid=(N // BLK,),
        in_specs=[
            pl.BlockSpec((NC, BLK, H), lambda i: (0, i, 0)),
            pl.BlockSpec((BLK, H), lambda i: (i, 0)),
            pl.BlockSpec((BLK, 1), lambda i: (i, 0)),
            pl.BlockSpec((1, H), lambda i: (0, 0)),
            pl.BlockSpec((H, H), lambda i: (0, 0)),
        ],
        out_specs=pl.BlockSpec((BLK, H), lambda i: (i, 0)),
        out_shape=jax.ShapeDtypeStruct((N, H), jnp.float32),
    )(aggp, g1, dinv, b1, w2)


def _tc3_body(aggp_ref, g2_ref, dinv_ref, b2_ref, wl_ref, bl_ref, out_ref):
    agg = aggp_ref[0] + aggp_ref[1] + g2_ref[...]
    h = jnp.maximum(agg * dinv_ref[...] + b2_ref[...], 0.0)
    out_ref[...] = jnp.sum(h * wl_ref[...], axis=1, keepdims=True) + bl_ref[...]


def _tc3(aggp, g2, dinv, b2, wl, bl):
    return pl.pallas_call(
        _tc3_body,
        grid=(N // BLK,),
        in_specs=[
            pl.BlockSpec((NC, BLK, H), lambda i: (0, i, 0)),
            pl.BlockSpec((BLK, H), lambda i: (i, 0)),
            pl.BlockSpec((BLK, 1), lambda i: (i, 0)),
            pl.BlockSpec((1, H), lambda i: (0, 0)),
            pl.BlockSpec((1, H), lambda i: (0, 0)),
            pl.BlockSpec((1, 1), lambda i: (0, 0)),
        ],
        out_specs=pl.BlockSpec((BLK, 1), lambda i: (i, 0)),
        out_shape=jax.ShapeDtypeStruct((N, 1), jnp.float32),
    )(aggp, g2, dinv, b2, wl, bl)


def kernel(x, edge_index, W1, b1, W2, b2, Wl, bl):
    pad_src = jnp.zeros((EPAD - E,), jnp.int32)
    pad_dst = jnp.full((EPAD - E,), NP - 1, jnp.int32)
    src = jnp.concatenate([edge_index[0], pad_src]).reshape(NC, NS, NCHUNK, CH)
    dst = jnp.concatenate([edge_index[1], pad_dst]).reshape(NC, NS, NCHUNK, CH)
    ones16 = jnp.ones((CH, 16), jnp.float32)
    zeros16 = jnp.zeros((NP, 16), jnp.float32)
    zerosh = jnp.zeros((NP, H), jnp.float32)

    degp = _sc_deg(dst, ones16, zeros16)
    deg2 = degp[:, :N, 0].T
    dinv, g1 = _tc1(deg2, x, W1)
    aggp1 = _sc_agg(g1, src, dst, zerosh)
    g2 = _tc2(aggp1, g1, dinv, b1.reshape(1, H), W2)
    aggp2 = _sc_agg(g2, src, dst, zerosh)
    out = _tc3(aggp2, g2, dinv, b2.reshape(1, H), Wl, bl.reshape(1, 1))
    return out.reshape(N)


# speedup vs baseline: 1.1298x; 1.1298x over previous
import jax
import jax.numpy as jnp
from jax import lax
from jax.experimental import pallas as pl
from jax.experimental.pallas import tpu as pltpu
from jax.experimental.pallas import tpu_sc as plsc

N = 10000
E = 320000
D = 128
H = 64

NC = 2
NS = 16
NW = NC * NS
CH = 128
NCHUNK = 80
EPAD = NW * NCHUNK * CH
NP = 10240
RPT = NP // NS
KD = 8

_sc_mesh = plsc.VectorSubcoreMesh(
    core_axis_name="c", subcore_axis_name="s", num_cores=NC, num_subcores=NS)


def _sc_deg_body(dst_hbm, ones_hbm, zeros_hbm, out_hbm, dstv, onesv, acc, sem):
    c = lax.axis_index("c")
    s = lax.axis_index("s")
    pltpu.sync_copy(dst_hbm.at[c, s], dstv)
    pltpu.sync_copy(ones_hbm, onesv)
    pltpu.sync_copy(zeros_hbm.at[pl.ds(s * RPT, RPT)],
                    acc.at[pl.ds(s * RPT, RPT)])
    plsc.subcore_barrier()

    def step(j, carry):
        pltpu.sync_copy(onesv, acc.at[dstv.at[j]], add=True)
        return carry

    lax.fori_loop(0, NCHUNK, step, 0)
    plsc.subcore_barrier()
    pltpu.sync_copy(acc.at[pl.ds(s * RPT, RPT)],
                    out_hbm.at[c, pl.ds(s * RPT, RPT)])


_sc_deg = pl.kernel(
    _sc_deg_body,
    out_type=jax.ShapeDtypeStruct((NC, NP, 16), jnp.float32),
    mesh=_sc_mesh,
    compiler_params=pltpu.CompilerParams(use_tc_tiling_on_sc=False),
    scratch_types=[
        pltpu.VMEM((NCHUNK, CH), jnp.int32),
        pltpu.VMEM((CH, 16), jnp.float32),
        pltpu.VMEM_SHARED((NP, 16), jnp.float32),
        pltpu.SemaphoreType.DMA,
    ],
)


def _sc_agg_body(g_hbm, src_hbm, dst_hbm, zeros_hbm, out_hbm,
                 srcv, dstv, rows, acc, sem):
    c = lax.axis_index("c")
    s = lax.axis_index("s")
    pltpu.sync_copy(src_hbm.at[c, s], srcv)
    pltpu.sync_copy(dst_hbm.at[c, s], dstv)
    pltpu.sync_copy(zeros_hbm.at[pl.ds(s * RPT, RPT)],
                    acc.at[pl.ds(s * RPT, RPT)])
    plsc.subcore_barrier()

    def group(i, carry):
        jj = i * KD
        descs = [
            pltpu.async_copy(g_hbm.at[srcv.at[jj + b]], rows.at[b], sem)
            for b in range(KD)
        ]
        for b in range(KD):
            descs[b].wait()
            pltpu.sync_copy(rows.at[b], acc.at[dstv.at[jj + b]], add=True)
        return carry

    lax.fori_loop(0, NCHUNK // KD, group, 0)
    plsc.subcore_barrier()
    pltpu.sync_copy(acc.at[pl.ds(s * RPT, RPT)],
                    out_hbm.at[c, pl.ds(s * RPT, RPT)])


_sc_agg = pl.kernel(
    _sc_agg_body,
    out_type=jax.ShapeDtypeStruct((NC, NP, H), jnp.float32),
    mesh=_sc_mesh,
    compiler_params=pltpu.CompilerParams(use_tc_tiling_on_sc=False),
    scratch_types=[
        pltpu.VMEM((NCHUNK, CH), jnp.int32),
        pltpu.VMEM((NCHUNK, CH), jnp.int32),
        pltpu.VMEM((KD, CH, H), jnp.float32),
        pltpu.VMEM_SHARED((NP, H), jnp.float32),
        pltpu.SemaphoreType.DMA,
    ],
)


BLK = 2000


def _tc1_body(deg_ref, x_ref, w1_ref, dinv_ref, g1_ref):
    deg = deg_ref[:, 0] + deg_ref[:, 1] + 1.0
    dinv = lax.rsqrt(deg)[:, None]
    h = lax.dot_general(x_ref[...], w1_ref[...], (((1,), (1,)), ((), ())),
                        preferred_element_type=jnp.float32)
    dinv_ref[...] = dinv
    g1_ref[...] = h * dinv


def _tc1(deg2, x, w1):
    return pl.pallas_call(
        _tc1_body,
        grid=(N // BLK,),
        in_specs=[
            pl.BlockSpec((BLK, 2), lambda i: (i, 0)),
            pl.BlockSpec((BLK, D), lambda i: (i, 0)),
            pl.BlockSpec((H, D), lambda i: (0, 0)),
        ],
        out_specs=[
            pl.BlockSpec((BLK, 1), lambda i: (i, 0)),
            pl.BlockSpec((BLK, H), lambda i: (i, 0)),
        ],
        out_shape=[
            jax.ShapeDtypeStruct((N, 1), jnp.float32),
            jax.ShapeDtypeStruct((N, H), jnp.float32),
        ],
    )(deg2, x, w1)


def _tc2_body(aggp_ref, g1_ref, dinv_ref, b1_ref, w2_ref, g2_ref):
    agg = aggp_ref[0] + aggp_ref[1] + g1_ref[...]
    h = jnp.maximum(agg * dinv_ref[...] + b1_ref[...], 0.0)
    g2 = lax.dot_general(h, w2_ref[...], (((1,), (1,)), ((), ())),
                         preferred_element_type=jnp.float32)
    g2_ref[...] = g2 * dinv_ref[...]


def _tc2(aggp, g1, dinv, b1, w2):
    return pl.pallas_call(
        _tc2_body,
        grid=(N // BLK,),
        in_specs=[
            pl.BlockSpec((NC, BLK, H), lambda i: (0, i, 0)),
            pl.BlockSpec((BLK, H), lambda i: (i, 0)),
            pl.BlockSpec((BLK, 1), lambda i: (i, 0)),
            pl.BlockSpec((1, H), lambda i: (0, 0)),
            pl.BlockSpec((H, H), lambda i: (0, 0)),
        ],
        out_specs=pl.BlockSpec((BLK, H), lambda i: (i, 0)),
        out_shape=jax.ShapeDtypeStruct((N, H), jnp.float32),
    )(aggp, g1, dinv, b1, w2)


def _tc3_body(aggp_ref, g2_ref, dinv_ref, b2_ref, wl_ref, bl_ref, out_ref):
    agg = aggp_ref[0] + aggp_ref[1] + g2_ref[...]
    h = jnp.maximum(agg * dinv_ref[...] + b2_ref[...], 0.0)
    out_ref[...] = jnp.sum(h * wl_ref[...], axis=1, keepdims=True) + bl_ref[...]


def _tc3(aggp, g2, dinv, b2, wl, bl):
    return pl.pallas_call(
        _tc3_body,
        grid=(N // BLK,),
        in_specs=[
            pl.BlockSpec((NC, BLK, H), lambda i: (0, i, 0)),
            pl.BlockSpec((BLK, H), lambda i: (i, 0)),
            pl.BlockSpec((BLK, 1), lambda i: (i, 0)),
            pl.BlockSpec((1, H), lambda i: (0, 0)),
            pl.BlockSpec((1, H), lambda i: (0, 0)),
            pl.BlockSpec((1, 1), lambda i: (0, 0)),
        ],
        out_specs=pl.BlockSpec((BLK, 1), lambda i: (i, 0)),
        out_shape=jax.ShapeDtypeStruct((N, 1), jnp.float32),
    )(aggp, g2, dinv, b2, wl, bl)


def kernel(x, edge_index, W1, b1, W2, b2, Wl, bl):
    pad_src = jnp.zeros((EPAD - E,), jnp.int32)
    pad_dst = N + jnp.arange(EPAD - E, dtype=jnp.int32) % (NP - N)
    src = jnp.concatenate([edge_index[0], pad_src]).reshape(NC, NS, NCHUNK, CH)
    dst = jnp.concatenate([edge_index[1], pad_dst]).reshape(NC, NS, NCHUNK, CH)
    ones16 = jnp.ones((CH, 16), jnp.float32)
    zeros16 = jnp.zeros((NP, 16), jnp.float32)
    zerosh = jnp.zeros((NP, H), jnp.float32)

    degp = _sc_deg(dst, ones16, zeros16)
    deg2 = degp[:, :N, 0].T
    dinv, g1 = _tc1(deg2, x, W1)
    aggp1 = _sc_agg(g1, src, dst, zerosh)
    g2 = _tc2(aggp1, g1, dinv, b1.reshape(1, H), W2)
    aggp2 = _sc_agg(g2, src, dst, zerosh)
    out = _tc3(aggp2, g2, dinv, b2.reshape(1, H), Wl, bl.reshape(1, 1))
    return out.reshape(N)


# speedup vs baseline: 2.2739x; 2.0126x over previous
import jax
import jax.numpy as jnp
from jax import lax
from jax.experimental import pallas as pl
from jax.experimental.pallas import tpu as pltpu
from jax.experimental.pallas import tpu_sc as plsc

N = 10000
E = 320000
D = 128
H = 64

NC = 2
NS = 16
NW = NC * NS
CH = 80
NCHUNK = 125
EPAD = NW * NCHUNK * CH
NP = 10240
RPT = NP // NS
KD = 5

_sc_mesh = plsc.VectorSubcoreMesh(
    core_axis_name="c", subcore_axis_name="s", num_cores=NC, num_subcores=NS)


def _sc_deg_body(dst_hbm, ones_hbm, zeros_hbm, out_hbm, dstv, onesv, acc, sem):
    c = lax.axis_index("c")
    s = lax.axis_index("s")
    pltpu.sync_copy(dst_hbm.at[c, s], dstv)
    pltpu.sync_copy(ones_hbm, onesv)
    pltpu.sync_copy(zeros_hbm.at[pl.ds(s * RPT, RPT)],
                    acc.at[pl.ds(s * RPT, RPT)])
    plsc.subcore_barrier()

    def step(j, carry):
        pltpu.sync_copy(onesv, acc.at[dstv.at[j]], add=True)
        return carry

    lax.fori_loop(0, NCHUNK, step, 0)
    plsc.subcore_barrier()
    pltpu.sync_copy(acc.at[pl.ds(s * RPT, RPT)],
                    out_hbm.at[c, pl.ds(s * RPT, RPT)])


_sc_deg = pl.kernel(
    _sc_deg_body,
    out_type=jax.ShapeDtypeStruct((NC, NP, 16), jnp.float32),
    mesh=_sc_mesh,
    compiler_params=pltpu.CompilerParams(use_tc_tiling_on_sc=False),
    scratch_types=[
        pltpu.VMEM((NCHUNK, CH), jnp.int32),
        pltpu.VMEM((CH, 16), jnp.float32),
        pltpu.VMEM_SHARED((NP, 16), jnp.float32),
        pltpu.SemaphoreType.DMA,
    ],
)


def _sc_agg_body(g_hbm, src_hbm, dst_hbm, zeros_hbm, out_hbm,
                 srcv, dstv, rows, acc, sem):
    c = lax.axis_index("c")
    s = lax.axis_index("s")
    pltpu.sync_copy(src_hbm.at[c, s], srcv)
    pltpu.sync_copy(dst_hbm.at[c, s], dstv)
    pltpu.sync_copy(zeros_hbm.at[pl.ds(s * RPT, RPT)],
                    acc.at[pl.ds(s * RPT, RPT)])
    plsc.subcore_barrier()

    def group(i, carry):
        jj = i * KD
        descs = [
            pltpu.async_copy(g_hbm.at[srcv.at[jj + b]], rows.at[b], sem)
            for b in range(KD)
        ]
        for b in range(KD):
            descs[b].wait()
            pltpu.sync_copy(rows.at[b], acc.at[dstv.at[jj + b]], add=True)
        return carry

    lax.fori_loop(0, NCHUNK // KD, group, 0)
    plsc.subcore_barrier()
    pltpu.sync_copy(acc.at[pl.ds(s * RPT, RPT)],
                    out_hbm.at[c, pl.ds(s * RPT, RPT)])


_sc_agg = pl.kernel(
    _sc_agg_body,
    out_type=jax.ShapeDtypeStruct((NC, NP, H), jnp.float32),
    mesh=_sc_mesh,
    compiler_params=pltpu.CompilerParams(use_tc_tiling_on_sc=False),
    scratch_types=[
        pltpu.VMEM((NCHUNK, CH), jnp.int32),
        pltpu.VMEM((NCHUNK, CH), jnp.int32),
        pltpu.VMEM((KD, CH, H), jnp.float32),
        pltpu.VMEM_SHARED((NP, H), jnp.float32),
        pltpu.SemaphoreType.DMA,
    ],
)


BLK = 2000


def _tc1_body(deg_ref, x_ref, w1_ref, dinv_ref, g1_ref):
    deg = deg_ref[:, 0] + deg_ref[:, 1] + 1.0
    dinv = lax.rsqrt(deg)[:, None]
    h = lax.dot_general(x_ref[...], w1_ref[...], (((1,), (1,)), ((), ())),
                        preferred_element_type=jnp.float32)
    dinv_ref[...] = dinv
    g1_ref[...] = h * dinv


def _tc1(deg2, x, w1):
    return pl.pallas_call(
        _tc1_body,
        grid=(N // BLK,),
        in_specs=[
            pl.BlockSpec((BLK, 2), lambda i: (i, 0)),
            pl.BlockSpec((BLK, D), lambda i: (i, 0)),
            pl.BlockSpec((H, D), lambda i: (0, 0)),
        ],
        out_specs=[
            pl.BlockSpec((BLK, 1), lambda i: (i, 0)),
            pl.BlockSpec((BLK, H), lambda i: (i, 0)),
        ],
        out_shape=[
            jax.ShapeDtypeStruct((N, 1), jnp.float32),
            jax.ShapeDtypeStruct((N, H), jnp.float32),
        ],
    )(deg2, x, w1)


def _tc2_body(aggp_ref, g1_ref, dinv_ref, b1_ref, w2_ref, g2_ref):
    agg = aggp_ref[0] + aggp_ref[1] + g1_ref[...]
    h = jnp.maximum(agg * dinv_ref[...] + b1_ref[...], 0.0)
    g2 = lax.dot_general(h, w2_ref[...], (((1,), (1,)), ((), ())),
                         preferred_element_type=jnp.float32)
    g2_ref[...] = g2 * dinv_ref[...]


def _tc2(aggp, g1, dinv, b1, w2):
    return pl.pallas_call(
        _tc2_body,
        grid=(N // BLK,),
        in_specs=[
            pl.BlockSpec((NC, BLK, H), lambda i: (0, i, 0)),
            pl.BlockSpec((BLK, H), lambda i: (i, 0)),
            pl.BlockSpec((BLK, 1), lambda i: (i, 0)),
            pl.BlockSpec((1, H), lambda i: (0, 0)),
            pl.BlockSpec((H, H), lambda i: (0, 0)),
        ],
        out_specs=pl.BlockSpec((BLK, H), lambda i: (i, 0)),
        out_shape=jax.ShapeDtypeStruct((N, H), jnp.float32),
    )(aggp, g1, dinv, b1, w2)


def _tc3_body(aggp_ref, g2_ref, dinv_ref, b2_ref, wl_ref, bl_ref, out_ref):
    agg = aggp_ref[0] + aggp_ref[1] + g2_ref[...]
    h = jnp.maximum(agg * dinv_ref[...] + b2_ref[...], 0.0)
    out_ref[...] = jnp.sum(h * wl_ref[...], axis=1, keepdims=True) + bl_ref[...]


def _tc3(aggp, g2, dinv, b2, wl, bl):
    return pl.pallas_call(
        _tc3_body,
        grid=(N // BLK,),
        in_specs=[
            pl.BlockSpec((NC, BLK, H), lambda i: (0, i, 0)),
            pl.BlockSpec((BLK, H), lambda i: (i, 0)),
            pl.BlockSpec((BLK, 1), lambda i: (i, 0)),
            pl.BlockSpec((1, H), lambda i: (0, 0)),
            pl.BlockSpec((1, H), lambda i: (0, 0)),
            pl.BlockSpec((1, 1), lambda i: (0, 0)),
        ],
        out_specs=pl.BlockSpec((BLK, 1), lambda i: (i, 0)),
        out_shape=jax.ShapeDtypeStruct((N, 1), jnp.float32),
    )(aggp, g2, dinv, b2, wl, bl)


def kernel(x, edge_index, W1, b1, W2, b2, Wl, bl):
    src = edge_index[0].reshape(NC, NS, NCHUNK, CH)
    dst = edge_index[1].reshape(NC, NS, NCHUNK, CH)
    ones16 = jnp.ones((CH, 16), jnp.float32)
    zeros16 = jnp.zeros((NP, 16), jnp.float32)
    zerosh = jnp.zeros((NP, H), jnp.float32)

    degp = _sc_deg(dst, ones16, zeros16)
    deg2 = degp[:, :N, 0].T
    dinv, g1 = _tc1(deg2, x, W1)
    aggp1 = _sc_agg(g1, src, dst, zerosh)
    g2 = _tc2(aggp1, g1, dinv, b1.reshape(1, H), W2)
    aggp2 = _sc_agg(g2, src, dst, zerosh)
    out = _tc3(aggp2, g2, dinv, b2.reshape(1, H), Wl, bl.reshape(1, 1))
    return out.reshape(N)


# speedup vs baseline: 3.3633x; 1.4791x over previous
import jax
import jax.numpy as jnp
from jax import lax
from jax.experimental import pallas as pl
from jax.experimental.pallas import tpu as pltpu
from jax.experimental.pallas import tpu_sc as plsc

N = 10000
E = 320000
D = 128
H = 64

NC = 2
NS = 16
NW = NC * NS
CH = 80
NCHUNK = 125
NP = 10240
RPT = NP // NS
KD = 5

_sc_mesh = plsc.VectorSubcoreMesh(
    core_axis_name="c", subcore_axis_name="s", num_cores=NC, num_subcores=NS)


def _sc_deg_body(edge_hbm, ones_hbm, zeros_hbm, out_hbm,
                 dstv, onesv, dbuf, obuf, acc, sem):
    c = lax.axis_index("c")
    s = lax.axis_index("s")
    pltpu.sync_copy(ones_hbm, onesv)
    pltpu.sync_copy(zeros_hbm.at[pl.ds(s * RPT, RPT)],
                    acc.at[pl.ds(s * RPT, RPT)])
    plsc.subcore_barrier()

    for c2 in range(NC):
        pltpu.sync_copy(edge_hbm.at[1, c2, s], dstv)

        def group(i, carry):
            jj = i * KD
            descs = [
                pltpu.async_copy(onesv, acc.at[dstv.at[jj + b]], sem,
                                 add=True)
                for b in range(KD)
            ]
            for d in descs:
                d.wait()
            return carry

        lax.fori_loop(0, NCHUNK // KD, group, 0)
    plsc.subcore_barrier()

    pltpu.sync_copy(acc.at[pl.ds(s * RPT, RPT)], dbuf)

    def post(r, carry):
        d = dbuf[r, :] + 1.0
        i = plsc.bitcast(d, jnp.int32)
        y = plsc.bitcast(0x5F3759DF - (i >> 1), jnp.float32)
        for _ in range(3):
            y = y * (1.5 - 0.5 * d * y * y)
        half = (r % 2) * H
        for k in range(4):
            obuf[r // 2, pl.ds(half + k * 16, 16)] = y
        return carry

    lax.fori_loop(0, RPT, post, 0)

    @pl.when(c == 0)
    def _():
        pltpu.sync_copy(obuf, out_hbm.at[pl.ds(s * (RPT // 2), RPT // 2)])


_sc_deg = pl.kernel(
    _sc_deg_body,
    out_type=jax.ShapeDtypeStruct((NP // 2, 2 * H), jnp.float32),
    mesh=_sc_mesh,
    compiler_params=pltpu.CompilerParams(use_tc_tiling_on_sc=False,
                                         needs_layout_passes=False),
    scratch_types=[
        pltpu.VMEM((NCHUNK, CH), jnp.int32),
        pltpu.VMEM((CH, 16), jnp.float32),
        pltpu.VMEM((RPT, 16), jnp.float32),
        pltpu.VMEM((RPT // 2, 2 * H), jnp.float32),
        pltpu.VMEM_SHARED((NP, 16), jnp.float32),
        pltpu.SemaphoreType.DMA,
    ],
)


def _sc_agg_body(g_hbm, edge_hbm, zeros_hbm, out_hbm,
                 srcv, dstv, rows, acc, sem):
    c = lax.axis_index("c")
    s = lax.axis_index("s")
    pltpu.sync_copy(edge_hbm.at[0, c, s], srcv)
    pltpu.sync_copy(edge_hbm.at[1, c, s], dstv)
    pltpu.sync_copy(zeros_hbm.at[pl.ds(s * RPT, RPT)],
                    acc.at[pl.ds(s * RPT, RPT)])
    plsc.subcore_barrier()

    def group(i, carry):
        jj = i * KD
        descs = [
            pltpu.async_copy(g_hbm.at[srcv.at[jj + b]], rows.at[b], sem)
            for b in range(KD)
        ]
        for b in range(KD):
            descs[b].wait()
            pltpu.sync_copy(rows.at[b], acc.at[dstv.at[jj + b]], add=True)
        return carry

    lax.fori_loop(0, NCHUNK // KD, group, 0)
    plsc.subcore_barrier()
    pltpu.sync_copy(acc.at[pl.ds(s * RPT, RPT)],
                    out_hbm.at[c, pl.ds(s * RPT, RPT)])


_sc_agg = pl.kernel(
    _sc_agg_body,
    out_type=jax.ShapeDtypeStruct((NC, NP, H), jnp.float32),
    mesh=_sc_mesh,
    compiler_params=pltpu.CompilerParams(use_tc_tiling_on_sc=False),
    scratch_types=[
        pltpu.VMEM((NCHUNK, CH), jnp.int32),
        pltpu.VMEM((NCHUNK, CH), jnp.int32),
        pltpu.VMEM((KD, CH, H), jnp.float32),
        pltpu.VMEM_SHARED((NP, H), jnp.float32),
        pltpu.SemaphoreType.DMA,
    ],
)


BLKP = 1000


def _tc1_body(xp_ref, m1_ref, dinv_ref, g1_ref):
    h = lax.dot_general(xp_ref[...], m1_ref[...], (((1,), (0,)), ((), ())),
                        preferred_element_type=jnp.float32)
    g1_ref[...] = h * dinv_ref[...]


def _tc1(xp, m1, dinv):
    return pl.pallas_call(
        _tc1_body,
        grid=(N // 2 // BLKP,),
        in_specs=[
            pl.BlockSpec((BLKP, 2 * D), lambda i: (i, 0)),
            pl.BlockSpec((2 * D, 2 * H), lambda i: (0, 0)),
            pl.BlockSpec((BLKP, 2 * H), lambda i: (i, 0)),
        ],
        out_specs=pl.BlockSpec((BLKP, 2 * H), lambda i: (i, 0)),
        out_shape=jax.ShapeDtypeStruct((N // 2, 2 * H), jnp.float32),
    )(xp, m1, dinv)


def _tc2_body(agg_ref, g1_ref, dinv_ref, b1_ref, m2_ref, g2_ref):
    a = agg_ref[0] + agg_ref[1] + g1_ref[...]
    h = jnp.maximum(a * dinv_ref[...] + b1_ref[...], 0.0)
    g2 = lax.dot_general(h, m2_ref[...], (((1,), (0,)), ((), ())),
                         preferred_element_type=jnp.float32)
    g2_ref[...] = g2 * dinv_ref[...]


def _tc2(aggv, g1p, dinv, b1p, m2):
    return pl.pallas_call(
        _tc2_body,
        grid=(N // 2 // BLKP,),
        in_specs=[
            pl.BlockSpec((NC, BLKP, 2 * H), lambda i: (0, i, 0)),
            pl.BlockSpec((BLKP, 2 * H), lambda i: (i, 0)),
            pl.BlockSpec((BLKP, 2 * H), lambda i: (i, 0)),
            pl.BlockSpec((1, 2 * H), lambda i: (0, 0)),
            pl.BlockSpec((2 * H, 2 * H), lambda i: (0, 0)),
        ],
        out_specs=pl.BlockSpec((BLKP, 2 * H), lambda i: (i, 0)),
        out_shape=jax.ShapeDtypeStruct((N // 2, 2 * H), jnp.float32),
    )(aggv, g1p, dinv, b1p, m2)


def _tc3_body(agg_ref, g2_ref, dinv_ref, b2_ref, wl_ref, bl_ref, out_ref):
    a = agg_ref[0] + agg_ref[1] + g2_ref[...]
    h = jnp.maximum(a * dinv_ref[...] + b2_ref[...], 0.0)
    t = h * wl_ref[...]
    o0 = jnp.sum(t[:, :H], axis=1, keepdims=True)
    o1 = jnp.sum(t[:, H:], axis=1, keepdims=True)
    out_ref[...] = jnp.concatenate([o0, o1], axis=1) + bl_ref[...]


def _tc3(aggv, g2p, dinv, b2p, wlp, bl):
    return pl.pallas_call(
        _tc3_body,
        grid=(N // 2 // BLKP,),
        in_specs=[
            pl.BlockSpec((NC, BLKP, 2 * H), lambda i: (0, i, 0)),
            pl.BlockSpec((BLKP, 2 * H), lambda i: (i, 0)),
            pl.BlockSpec((BLKP, 2 * H), lambda i: (i, 0)),
            pl.BlockSpec((1, 2 * H), lambda i: (0, 0)),
            pl.BlockSpec((1, 2 * H), lambda i: (0, 0)),
            pl.BlockSpec((1, 1), lambda i: (0, 0)),
        ],
        out_specs=pl.BlockSpec((BLKP, 2), lambda i: (i, 0)),
        out_shape=jax.ShapeDtypeStruct((N // 2, 2), jnp.float32),
    )(aggv, g2p, dinv, b2p, wlp, bl)


def kernel(x, edge_index, W1, b1, W2, b2, Wl, bl):
    f32 = jnp.float32
    edge4 = edge_index.reshape(2, NC, NS, NCHUNK, CH)
    ones16 = jnp.ones((CH, 16), f32)
    zeros16 = jnp.zeros((NP, 16), f32)
    zerosh = jnp.zeros((NP, H), f32)
    m1 = jnp.zeros((2 * D, 2 * H), f32)
    m1 = m1.at[:D, :H].set(W1.T).at[D:, H:].set(W1.T)
    m2 = jnp.zeros((2 * H, 2 * H), f32)
    m2 = m2.at[:H, :H].set(W2.T).at[H:, H:].set(W2.T)
    b1p = jnp.concatenate([b1, b1]).reshape(1, 2 * H)
    b2p = jnp.concatenate([b2, b2]).reshape(1, 2 * H)
    wlp = jnp.concatenate([Wl[0], Wl[0]]).reshape(1, 2 * H)

    dinvp = _sc_deg(edge4, ones16, zeros16)
    xp = x.reshape(N // 2, 2 * D)
    g1p = _tc1(xp, m1, dinvp)
    aggp1 = _sc_agg(g1p.reshape(N, H), edge4, zerosh)
    g2p = _tc2(aggp1.reshape(NC, NP // 2, 2 * H), g1p, dinvp, b1p, m2)
    aggp2 = _sc_agg(g2p.reshape(N, H), edge4, zerosh)
    out = _tc3(aggp2.reshape(NC, NP // 2, 2 * H), g2p, dinvp, b2p, wlp,
               bl.reshape(1, 1))
    return out.reshape(N)


# speedup vs baseline: 3.4585x; 1.0283x over previous
import jax
import jax.numpy as jnp
from jax import lax
from jax.experimental import pallas as pl
from jax.experimental.pallas import tpu as pltpu
from jax.experimental.pallas import tpu_sc as plsc

N = 10000
E = 320000
D = 128
H = 64

NC = 2
NS = 16
NW = NC * NS
CH = 80
NCHUNK = 125
NP = 10240
RPT = NP // NS
KD = 5

_sc_mesh = plsc.VectorSubcoreMesh(
    core_axis_name="c", subcore_axis_name="s", num_cores=NC, num_subcores=NS)


def _sc_deg_body(edge_hbm, ones_hbm, zeros_hbm, out_hbm,
                 dstv, onesv, dbuf, obuf, acc, sem):
    c = lax.axis_index("c")
    s = lax.axis_index("s")
    pltpu.sync_copy(ones_hbm, onesv)
    pltpu.sync_copy(zeros_hbm.at[pl.ds(s * RPT, RPT)],
                    acc.at[pl.ds(s * RPT, RPT)])
    plsc.subcore_barrier()

    for c2 in range(NC):
        pltpu.sync_copy(edge_hbm.at[1, c2, s], dstv)

        def group(i, carry):
            jj = i * KD
            descs = [
                pltpu.async_copy(onesv, acc.at[dstv.at[jj + b]], sem,
                                 add=True)
                for b in range(KD)
            ]
            for d in descs:
                d.wait()
            return carry

        lax.fori_loop(0, NCHUNK // KD, group, 0)
    plsc.subcore_barrier()

    pltpu.sync_copy(acc.at[pl.ds(s * RPT, RPT)], dbuf)

    def post(r, carry):
        d = dbuf[r, :] + 1.0
        i = plsc.bitcast(d, jnp.int32)
        y = plsc.bitcast(0x5F3759DF - (i >> 1), jnp.float32)
        for _ in range(3):
            y = y * (1.5 - 0.5 * d * y * y)
        half = (r % 2) * H
        for k in range(4):
            obuf[r // 2, pl.ds(half + k * 16, 16)] = y
        return carry

    lax.fori_loop(0, RPT, post, 0)

    @pl.when(c == 0)
    def _():
        pltpu.sync_copy(obuf, out_hbm.at[pl.ds(s * (RPT // 2), RPT // 2)])


_sc_deg = pl.kernel(
    _sc_deg_body,
    out_type=jax.ShapeDtypeStruct((NP // 2, 2 * H), jnp.float32),
    mesh=_sc_mesh,
    compiler_params=pltpu.CompilerParams(use_tc_tiling_on_sc=False,
                                         needs_layout_passes=False),
    scratch_types=[
        pltpu.VMEM((NCHUNK, CH), jnp.int32),
        pltpu.VMEM((CH, 16), jnp.float32),
        pltpu.VMEM((RPT, 16), jnp.float32),
        pltpu.VMEM((RPT // 2, 2 * H), jnp.float32),
        pltpu.VMEM_SHARED((NP, 16), jnp.float32),
        pltpu.SemaphoreType.DMA,
    ],
)


def _sc_agg_body(g_hbm, edge_hbm, zeros_hbm, out_hbm,
                 srcv, dstv, rows, acc, gsem, ssem):
    c = lax.axis_index("c")
    s = lax.axis_index("s")
    pltpu.sync_copy(edge_hbm.at[0, c, s], srcv)
    pltpu.sync_copy(edge_hbm.at[1, c, s], dstv)
    pltpu.sync_copy(zeros_hbm.at[pl.ds(s * RPT, RPT)],
                    acc.at[pl.ds(s * RPT, RPT)])
    plsc.subcore_barrier()

    def group(i, carry):
        jj = i * KD
        gd = [
            pltpu.async_copy(g_hbm.at[srcv.at[jj + b]], rows.at[b], gsem)
            for b in range(KD)
        ]
        sd = []
        for b in range(KD):
            gd[b].wait()
            sd.append(pltpu.async_copy(rows.at[b], acc.at[dstv.at[jj + b]],
                                       ssem, add=True))
        for d in sd:
            d.wait()
        return carry

    lax.fori_loop(0, NCHUNK // KD, group, 0)
    plsc.subcore_barrier()
    pltpu.sync_copy(acc.at[pl.ds(s * RPT, RPT)],
                    out_hbm.at[c, pl.ds(s * RPT, RPT)])


_sc_agg = pl.kernel(
    _sc_agg_body,
    out_type=jax.ShapeDtypeStruct((NC, NP, H), jnp.float32),
    mesh=_sc_mesh,
    compiler_params=pltpu.CompilerParams(use_tc_tiling_on_sc=False),
    scratch_types=[
        pltpu.VMEM((NCHUNK, CH), jnp.int32),
        pltpu.VMEM((NCHUNK, CH), jnp.int32),
        pltpu.VMEM((KD, CH, H), jnp.float32),
        pltpu.VMEM_SHARED((NP, H), jnp.float32),
        pltpu.SemaphoreType.DMA,
        pltpu.SemaphoreType.DMA,
    ],
)


BLKP = 1000


def _tc1_body(xp_ref, m1_ref, dinv_ref, g1_ref):
    h = lax.dot_general(xp_ref[...], m1_ref[...], (((1,), (0,)), ((), ())),
                        preferred_element_type=jnp.float32)
    g1_ref[...] = h * dinv_ref[...]


def _tc1(xp, m1, dinv):
    return pl.pallas_call(
        _tc1_body,
        grid=(N // 2 // BLKP,),
        in_specs=[
            pl.BlockSpec((BLKP, 2 * D), lambda i: (i, 0)),
            pl.BlockSpec((2 * D, 2 * H), lambda i: (0, 0)),
            pl.BlockSpec((BLKP, 2 * H), lambda i: (i, 0)),
        ],
        out_specs=pl.BlockSpec((BLKP, 2 * H), lambda i: (i, 0)),
        out_shape=jax.ShapeDtypeStruct((N // 2, 2 * H), jnp.float32),
    )(xp, m1, dinv)


def _tc2_body(agg_ref, g1_ref, dinv_ref, b1_ref, m2_ref, g2_ref):
    a = agg_ref[0] + agg_ref[1] + g1_ref[...]
    h = jnp.maximum(a * dinv_ref[...] + b1_ref[...], 0.0)
    g2 = lax.dot_general(h, m2_ref[...], (((1,), (0,)), ((), ())),
                         preferred_element_type=jnp.float32)
    g2_ref[...] = g2 * dinv_ref[...]


def _tc2(aggv, g1p, dinv, b1p, m2):
    return pl.pallas_call(
        _tc2_body,
        grid=(N // 2 // BLKP,),
        in_specs=[
            pl.BlockSpec((NC, BLKP, 2 * H), lambda i: (0, i, 0)),
            pl.BlockSpec((BLKP, 2 * H), lambda i: (i, 0)),
            pl.BlockSpec((BLKP, 2 * H), lambda i: (i, 0)),
            pl.BlockSpec((1, 2 * H), lambda i: (0, 0)),
            pl.BlockSpec((2 * H, 2 * H), lambda i: (0, 0)),
        ],
        out_specs=pl.BlockSpec((BLKP, 2 * H), lambda i: (i, 0)),
        out_shape=jax.ShapeDtypeStruct((N // 2, 2 * H), jnp.float32),
    )(aggv, g1p, dinv, b1p, m2)


def _tc3_body(agg_ref, g2_ref, dinv_ref, b2_ref, wl_ref, bl_ref, out_ref):
    a = agg_ref[0] + agg_ref[1] + g2_ref[...]
    h = jnp.maximum(a * dinv_ref[...] + b2_ref[...], 0.0)
    t = h * wl_ref[...]
    o0 = jnp.sum(t[:, :H], axis=1, keepdims=True)
    o1 = jnp.sum(t[:, H:], axis=1, keepdims=True)
    out_ref[...] = jnp.concatenate([o0, o1], axis=1) + bl_ref[...]


def _tc3(aggv, g2p, dinv, b2p, wlp, bl):
    return pl.pallas_call(
        _tc3_body,
        grid=(N // 2 // BLKP,),
        in_specs=[
            pl.BlockSpec((NC, BLKP, 2 * H), lambda i: (0, i, 0)),
            pl.BlockSpec((BLKP, 2 * H), lambda i: (i, 0)),
            pl.BlockSpec((BLKP, 2 * H), lambda i: (i, 0)),
            pl.BlockSpec((1, 2 * H), lambda i: (0, 0)),
            pl.BlockSpec((1, 2 * H), lambda i: (0, 0)),
            pl.BlockSpec((1, 1), lambda i: (0, 0)),
        ],
        out_specs=pl.BlockSpec((BLKP, 2), lambda i: (i, 0)),
        out_shape=jax.ShapeDtypeStruct((N // 2, 2), jnp.float32),
    )(aggv, g2p, dinv, b2p, wlp, bl)


def kernel(x, edge_index, W1, b1, W2, b2, Wl, bl):
    f32 = jnp.float32
    edge4 = edge_index.reshape(2, NC, NS, NCHUNK, CH)
    ones16 = jnp.ones((CH, 16), f32)
    zeros16 = jnp.zeros((NP, 16), f32)
    zerosh = jnp.zeros((NP, H), f32)
    m1 = jnp.zeros((2 * D, 2 * H), f32)
    m1 = m1.at[:D, :H].set(W1.T).at[D:, H:].set(W1.T)
    m2 = jnp.zeros((2 * H, 2 * H), f32)
    m2 = m2.at[:H, :H].set(W2.T).at[H:, H:].set(W2.T)
    b1p = jnp.concatenate([b1, b1]).reshape(1, 2 * H)
    b2p = jnp.concatenate([b2, b2]).reshape(1, 2 * H)
    wlp = jnp.concatenate([Wl[0], Wl[0]]).reshape(1, 2 * H)

    dinvp = _sc_deg(edge4, ones16, zeros16)
    xp = x.reshape(N // 2, 2 * D)
    g1p = _tc1(xp, m1, dinvp)
    aggp1 = _sc_agg(g1p.reshape(N, H), edge4, zerosh)
    g2p = _tc2(aggp1.reshape(NC, NP // 2, 2 * H), g1p, dinvp, b1p, m2)
    aggp2 = _sc_agg(g2p.reshape(N, H), edge4, zerosh)
    out = _tc3(aggp2.reshape(NC, NP // 2, 2 * H), g2p, dinvp, b2p, wlp,
               bl.reshape(1, 1))
    return out.reshape(N)


# speedup vs baseline: 3.5001x; 1.0120x over previous
import jax
import jax.numpy as jnp
from jax import lax
from jax.experimental import pallas as pl
from jax.experimental.pallas import tpu as pltpu
from jax.experimental.pallas import tpu_sc as plsc

N = 10000
E = 320000
D = 128
H = 64

NC = 2
NS = 16
NW = NC * NS
CH = 40
NCHUNK = 250
NP = 10240
RPT = NP // NS
KD = 10

_sc_mesh = plsc.VectorSubcoreMesh(
    core_axis_name="c", subcore_axis_name="s", num_cores=NC, num_subcores=NS)


def _sc_deg_body(edge_hbm, ones_hbm, zeros_hbm, out_hbm,
                 dstv, onesv, dbuf, obuf, acc, sem):
    c = lax.axis_index("c")
    s = lax.axis_index("s")
    pltpu.sync_copy(ones_hbm, onesv)
    pltpu.sync_copy(zeros_hbm.at[pl.ds(s * RPT, RPT)],
                    acc.at[pl.ds(s * RPT, RPT)])
    plsc.subcore_barrier()

    for c2 in range(NC):
        pltpu.sync_copy(edge_hbm.at[1, c2, s], dstv)

        def group(i, carry):
            jj = i * KD
            descs = [
                pltpu.async_copy(onesv, acc.at[dstv.at[jj + b]], sem,
                                 add=True)
                for b in range(KD)
            ]
            for d in descs:
                d.wait()
            return carry

        lax.fori_loop(0, NCHUNK // KD, group, 0)
    plsc.subcore_barrier()

    pltpu.sync_copy(acc.at[pl.ds(s * RPT, RPT)], dbuf)

    def post(r, carry):
        d = dbuf[r, :] + 1.0
        i = plsc.bitcast(d, jnp.int32)
        y = plsc.bitcast(0x5F3759DF - (i >> 1), jnp.float32)
        for _ in range(3):
            y = y * (1.5 - 0.5 * d * y * y)
        half = (r % 2) * H
        for k in range(4):
            obuf[r // 2, pl.ds(half + k * 16, 16)] = y
        return carry

    lax.fori_loop(0, RPT, post, 0)

    @pl.when(c == 0)
    def _():
        pltpu.sync_copy(obuf, out_hbm.at[pl.ds(s * (RPT // 2), RPT // 2)])


_sc_deg = pl.kernel(
    _sc_deg_body,
    out_type=jax.ShapeDtypeStruct((NP // 2, 2 * H), jnp.float32),
    mesh=_sc_mesh,
    compiler_params=pltpu.CompilerParams(use_tc_tiling_on_sc=False,
                                         needs_layout_passes=False),
    scratch_types=[
        pltpu.VMEM((NCHUNK, CH), jnp.int32),
        pltpu.VMEM((CH, 16), jnp.float32),
        pltpu.VMEM((RPT, 16), jnp.float32),
        pltpu.VMEM((RPT // 2, 2 * H), jnp.float32),
        pltpu.VMEM_SHARED((NP, 16), jnp.float32),
        pltpu.SemaphoreType.DMA,
    ],
)


def _sc_agg_body(g_hbm, edge_hbm, zeros_hbm, out_hbm,
                 srcv, dstv, rows, acc, gsem, ssem):
    c = lax.axis_index("c")
    s = lax.axis_index("s")
    pltpu.sync_copy(edge_hbm.at[0, c, s], srcv)
    pltpu.sync_copy(edge_hbm.at[1, c, s], dstv)
    pltpu.sync_copy(zeros_hbm.at[pl.ds(s * RPT, RPT)],
                    acc.at[pl.ds(s * RPT, RPT)])
    plsc.subcore_barrier()

    def group(i, carry):
        jj = i * KD
        gd = [
            pltpu.async_copy(g_hbm.at[srcv.at[jj + b]], rows.at[b], gsem)
            for b in range(KD)
        ]
        sd = []
        for b in range(KD):
            gd[b].wait()
            sd.append(pltpu.async_copy(rows.at[b], acc.at[dstv.at[jj + b]],
                                       ssem, add=True))
        for d in sd:
            d.wait()
        return carry

    lax.fori_loop(0, NCHUNK // KD, group, 0)
    plsc.subcore_barrier()
    pltpu.sync_copy(acc.at[pl.ds(s * RPT, RPT)],
                    out_hbm.at[c, pl.ds(s * RPT, RPT)])


_sc_agg = pl.kernel(
    _sc_agg_body,
    out_type=jax.ShapeDtypeStruct((NC, NP, H), jnp.float32),
    mesh=_sc_mesh,
    compiler_params=pltpu.CompilerParams(use_tc_tiling_on_sc=False),
    scratch_types=[
        pltpu.VMEM((NCHUNK, CH), jnp.int32),
        pltpu.VMEM((NCHUNK, CH), jnp.int32),
        pltpu.VMEM((KD, CH, H), jnp.float32),
        pltpu.VMEM_SHARED((NP, H), jnp.float32),
        pltpu.SemaphoreType.DMA,
        pltpu.SemaphoreType.DMA,
    ],
)


BLKP = 1000


def _tc1_body(xp_ref, m1_ref, dinv_ref, g1_ref):
    h = lax.dot_general(xp_ref[...], m1_ref[...], (((1,), (0,)), ((), ())),
                        preferred_element_type=jnp.float32)
    g1_ref[...] = h * dinv_ref[...]


def _tc1(xp, m1, dinv):
    return pl.pallas_call(
        _tc1_body,
        grid=(N // 2 // BLKP,),
        in_specs=[
            pl.BlockSpec((BLKP, 2 * D), lambda i: (i, 0)),
            pl.BlockSpec((2 * D, 2 * H), lambda i: (0, 0)),
            pl.BlockSpec((BLKP, 2 * H), lambda i: (i, 0)),
        ],
        out_specs=pl.BlockSpec((BLKP, 2 * H), lambda i: (i, 0)),
        out_shape=jax.ShapeDtypeStruct((N // 2, 2 * H), jnp.float32),
    )(xp, m1, dinv)


def _tc2_body(agg_ref, g1_ref, dinv_ref, b1_ref, m2_ref, g2_ref):
    a = agg_ref[0] + agg_ref[1] + g1_ref[...]
    h = jnp.maximum(a * dinv_ref[...] + b1_ref[...], 0.0)
    g2 = lax.dot_general(h, m2_ref[...], (((1,), (0,)), ((), ())),
                         preferred_element_type=jnp.float32)
    g2_ref[...] = g2 * dinv_ref[...]


def _tc2(aggv, g1p, dinv, b1p, m2):
    return pl.pallas_call(
        _tc2_body,
        grid=(N // 2 // BLKP,),
        in_specs=[
            pl.BlockSpec((NC, BLKP, 2 * H), lambda i: (0, i, 0)),
            pl.BlockSpec((BLKP, 2 * H), lambda i: (i, 0)),
            pl.BlockSpec((BLKP, 2 * H), lambda i: (i, 0)),
            pl.BlockSpec((1, 2 * H), lambda i: (0, 0)),
            pl.BlockSpec((2 * H, 2 * H), lambda i: (0, 0)),
        ],
        out_specs=pl.BlockSpec((BLKP, 2 * H), lambda i: (i, 0)),
        out_shape=jax.ShapeDtypeStruct((N // 2, 2 * H), jnp.float32),
    )(aggv, g1p, dinv, b1p, m2)


def _tc3_body(agg_ref, g2_ref, dinv_ref, b2_ref, wl_ref, bl_ref, out_ref):
    a = agg_ref[0] + agg_ref[1] + g2_ref[...]
    h = jnp.maximum(a * dinv_ref[...] + b2_ref[...], 0.0)
    t = h * wl_ref[...]
    o0 = jnp.sum(t[:, :H], axis=1, keepdims=True)
    o1 = jnp.sum(t[:, H:], axis=1, keepdims=True)
    out_ref[...] = jnp.concatenate([o0, o1], axis=1) + bl_ref[...]


def _tc3(aggv, g2p, dinv, b2p, wlp, bl):
    return pl.pallas_call(
        _tc3_body,
        grid=(N // 2 // BLKP,),
        in_specs=[
            pl.BlockSpec((NC, BLKP, 2 * H), lambda i: (0, i, 0)),
            pl.BlockSpec((BLKP, 2 * H), lambda i: (i, 0)),
            pl.BlockSpec((BLKP, 2 * H), lambda i: (i, 0)),
            pl.BlockSpec((1, 2 * H), lambda i: (0, 0)),
            pl.BlockSpec((1, 2 * H), lambda i: (0, 0)),
            pl.BlockSpec((1, 1), lambda i: (0, 0)),
        ],
        out_specs=pl.BlockSpec((BLKP, 2), lambda i: (i, 0)),
        out_shape=jax.ShapeDtypeStruct((N // 2, 2), jnp.float32),
    )(aggv, g2p, dinv, b2p, wlp, bl)


def kernel(x, edge_index, W1, b1, W2, b2, Wl, bl):
    f32 = jnp.float32
    edge4 = edge_index.reshape(2, NC, NS, NCHUNK, CH)
    ones16 = jnp.ones((CH, 16), f32)
    zeros16 = jnp.zeros((NP, 16), f32)
    zerosh = jnp.zeros((NP, H), f32)
    m1 = jnp.zeros((2 * D, 2 * H), f32)
    m1 = m1.at[:D, :H].set(W1.T).at[D:, H:].set(W1.T)
    m2 = jnp.zeros((2 * H, 2 * H), f32)
    m2 = m2.at[:H, :H].set(W2.T).at[H:, H:].set(W2.T)
    b1p = jnp.concatenate([b1, b1]).reshape(1, 2 * H)
    b2p = jnp.concatenate([b2, b2]).reshape(1, 2 * H)
    wlp = jnp.concatenate([Wl[0], Wl[0]]).reshape(1, 2 * H)

    dinvp = _sc_deg(edge4, ones16, zeros16)
    xp = x.reshape(N // 2, 2 * D)
    g1p = _tc1(xp, m1, dinvp)
    aggp1 = _sc_agg(g1p.reshape(N, H), edge4, zerosh)
    g2p = _tc2(aggp1.reshape(NC, NP // 2, 2 * H), g1p, dinvp, b1p, m2)
    aggp2 = _sc_agg(g2p.reshape(N, H), edge4, zerosh)
    out = _tc3(aggp2.reshape(NC, NP // 2, 2 * H), g2p, dinvp, b2p, wlp,
               bl.reshape(1, 1))
    return out.reshape(N)


# speedup vs baseline: 3.6395x; 1.0398x over previous
import jax
import jax.numpy as jnp
from jax import lax
from jax.experimental import pallas as pl
from jax.experimental.pallas import tpu as pltpu
from jax.experimental.pallas import tpu_sc as plsc

N = 10000
E = 320000
D = 128
H = 64

NC = 2
NS = 16
NW = NC * NS
CH = 40
NCHUNK = 250
NP = 10240
RPT = NP // NS
KD = 25

_sc_mesh = plsc.VectorSubcoreMesh(
    core_axis_name="c", subcore_axis_name="s", num_cores=NC, num_subcores=NS)


def _sc_deg_body(edge_hbm, ones_hbm, zeros_hbm, out_hbm,
                 dstv, onesv, dbuf, obuf, acc, sem):
    c = lax.axis_index("c")
    s = lax.axis_index("s")
    pltpu.sync_copy(ones_hbm, onesv)
    pltpu.sync_copy(zeros_hbm.at[pl.ds(s * RPT, RPT)],
                    acc.at[pl.ds(s * RPT, RPT)])
    plsc.subcore_barrier()

    for c2 in range(NC):
        pltpu.sync_copy(edge_hbm.at[1, c2, s], dstv)

        def group(i, carry):
            jj = i * KD
            descs = [
                pltpu.async_copy(onesv, acc.at[dstv.at[jj + b]], sem,
                                 add=True)
                for b in range(KD)
            ]
            for d in descs:
                d.wait()
            return carry

        lax.fori_loop(0, NCHUNK // KD, group, 0)
    plsc.subcore_barrier()

    pltpu.sync_copy(acc.at[pl.ds(s * RPT, RPT)], dbuf)

    def post(r, carry):
        d = dbuf[r, :] + 1.0
        i = plsc.bitcast(d, jnp.int32)
        y = plsc.bitcast(0x5F3759DF - (i >> 1), jnp.float32)
        for _ in range(3):
            y = y * (1.5 - 0.5 * d * y * y)
        half = (r % 2) * H
        for k in range(4):
            obuf[r // 2, pl.ds(half + k * 16, 16)] = y
        return carry

    lax.fori_loop(0, RPT, post, 0)

    @pl.when(c == 0)
    def _():
        pltpu.sync_copy(obuf, out_hbm.at[pl.ds(s * (RPT // 2), RPT // 2)])


_sc_deg = pl.kernel(
    _sc_deg_body,
    out_type=jax.ShapeDtypeStruct((NP // 2, 2 * H), jnp.float32),
    mesh=_sc_mesh,
    compiler_params=pltpu.CompilerParams(use_tc_tiling_on_sc=False,
                                         needs_layout_passes=False),
    scratch_types=[
        pltpu.VMEM((NCHUNK, CH), jnp.int32),
        pltpu.VMEM((CH, 16), jnp.float32),
        pltpu.VMEM((RPT, 16), jnp.float32),
        pltpu.VMEM((RPT // 2, 2 * H), jnp.float32),
        pltpu.VMEM_SHARED((NP, 16), jnp.float32),
        pltpu.SemaphoreType.DMA,
    ],
)


def _sc_agg_body(g_hbm, edge_hbm, zeros_hbm, out_hbm,
                 srcv, dstv, rows, acc, gsem, ssem):
    c = lax.axis_index("c")
    s = lax.axis_index("s")
    pltpu.sync_copy(edge_hbm.at[0, c, s], srcv)
    pltpu.sync_copy(edge_hbm.at[1, c, s], dstv)
    pltpu.sync_copy(zeros_hbm.at[pl.ds(s * RPT, RPT)],
                    acc.at[pl.ds(s * RPT, RPT)])
    plsc.subcore_barrier()

    def group(i, carry):
        jj = i * KD
        gd = [
            pltpu.async_copy(g_hbm.at[srcv.at[jj + b]], rows.at[b], gsem)
            for b in range(KD)
        ]
        sd = []
        for b in range(KD):
            gd[b].wait()
            sd.append(pltpu.async_copy(rows.at[b], acc.at[dstv.at[jj + b]],
                                       ssem, add=True))
        for d in sd:
            d.wait()
        return carry

    lax.fori_loop(0, NCHUNK // KD, group, 0)
    plsc.subcore_barrier()
    pltpu.sync_copy(acc.at[pl.ds(s * RPT, RPT)],
                    out_hbm.at[c, pl.ds(s * RPT, RPT)])


_sc_agg = pl.kernel(
    _sc_agg_body,
    out_type=jax.ShapeDtypeStruct((NC, NP, H), jnp.float32),
    mesh=_sc_mesh,
    compiler_params=pltpu.CompilerParams(use_tc_tiling_on_sc=False),
    scratch_types=[
        pltpu.VMEM((NCHUNK, CH), jnp.int32),
        pltpu.VMEM((NCHUNK, CH), jnp.int32),
        pltpu.VMEM((KD, CH, H), jnp.float32),
        pltpu.VMEM_SHARED((NP, H), jnp.float32),
        pltpu.SemaphoreType.DMA,
        pltpu.SemaphoreType.DMA,
    ],
)


BLKP = 1000


def _tc1_body(xp_ref, m1_ref, dinv_ref, g1_ref):
    h = lax.dot_general(xp_ref[...], m1_ref[...], (((1,), (0,)), ((), ())),
                        preferred_element_type=jnp.float32)
    g1_ref[...] = h * dinv_ref[...]


def _tc1(xp, m1, dinv):
    return pl.pallas_call(
        _tc1_body,
        grid=(N // 2 // BLKP,),
        in_specs=[
            pl.BlockSpec((BLKP, 2 * D), lambda i: (i, 0)),
            pl.BlockSpec((2 * D, 2 * H), lambda i: (0, 0)),
            pl.BlockSpec((BLKP, 2 * H), lambda i: (i, 0)),
        ],
        out_specs=pl.BlockSpec((BLKP, 2 * H), lambda i: (i, 0)),
        out_shape=jax.ShapeDtypeStruct((N // 2, 2 * H), jnp.float32),
    )(xp, m1, dinv)


def _tc2_body(agg_ref, g1_ref, dinv_ref, b1_ref, m2_ref, g2_ref):
    a = agg_ref[0] + agg_ref[1] + g1_ref[...]
    h = jnp.maximum(a * dinv_ref[...] + b1_ref[...], 0.0)
    g2 = lax.dot_general(h, m2_ref[...], (((1,), (0,)), ((), ())),
                         preferred_element_type=jnp.float32)
    g2_ref[...] = g2 * dinv_ref[...]


def _tc2(aggv, g1p, dinv, b1p, m2):
    return pl.pallas_call(
        _tc2_body,
        grid=(N // 2 // BLKP,),
        in_specs=[
            pl.BlockSpec((NC, BLKP, 2 * H), lambda i: (0, i, 0)),
            pl.BlockSpec((BLKP, 2 * H), lambda i: (i, 0)),
            pl.BlockSpec((BLKP, 2 * H), lambda i: (i, 0)),
            pl.BlockSpec((1, 2 * H), lambda i: (0, 0)),
            pl.BlockSpec((2 * H, 2 * H), lambda i: (0, 0)),
        ],
        out_specs=pl.BlockSpec((BLKP, 2 * H), lambda i: (i, 0)),
        out_shape=jax.ShapeDtypeStruct((N // 2, 2 * H), jnp.float32),
    )(aggv, g1p, dinv, b1p, m2)


def _tc3_body(agg_ref, g2_ref, dinv_ref, b2_ref, wl_ref, bl_ref, out_ref):
    a = agg_ref[0] + agg_ref[1] + g2_ref[...]
    h = jnp.maximum(a * dinv_ref[...] + b2_ref[...], 0.0)
    t = h * wl_ref[...]
    o0 = jnp.sum(t[:, :H], axis=1, keepdims=True)
    o1 = jnp.sum(t[:, H:], axis=1, keepdims=True)
    out_ref[...] = jnp.concatenate([o0, o1], axis=1) + bl_ref[...]


def _tc3(aggv, g2p, dinv, b2p, wlp, bl):
    return pl.pallas_call(
        _tc3_body,
        grid=(N // 2 // BLKP,),
        in_specs=[
            pl.BlockSpec((NC, BLKP, 2 * H), lambda i: (0, i, 0)),
            pl.BlockSpec((BLKP, 2 * H), lambda i: (i, 0)),
            pl.BlockSpec((BLKP, 2 * H), lambda i: (i, 0)),
            pl.BlockSpec((1, 2 * H), lambda i: (0, 0)),
            pl.BlockSpec((1, 2 * H), lambda i: (0, 0)),
            pl.BlockSpec((1, 1), lambda i: (0, 0)),
        ],
        out_specs=pl.BlockSpec((BLKP, 2), lambda i: (i, 0)),
        out_shape=jax.ShapeDtypeStruct((N // 2, 2), jnp.float32),
    )(aggv, g2p, dinv, b2p, wlp, bl)


def kernel(x, edge_index, W1, b1, W2, b2, Wl, bl):
    f32 = jnp.float32
    edge4 = edge_index.reshape(2, NC, NS, NCHUNK, CH)
    ones16 = jnp.ones((CH, 16), f32)
    zeros16 = jnp.zeros((NP, 16), f32)
    zerosh = jnp.zeros((NP, H), f32)
    m1 = jnp.zeros((2 * D, 2 * H), f32)
    m1 = m1.at[:D, :H].set(W1.T).at[D:, H:].set(W1.T)
    m2 = jnp.zeros((2 * H, 2 * H), f32)
    m2 = m2.at[:H, :H].set(W2.T).at[H:, H:].set(W2.T)
    b1p = jnp.concatenate([b1, b1]).reshape(1, 2 * H)
    b2p = jnp.concatenate([b2, b2]).reshape(1, 2 * H)
    wlp = jnp.concatenate([Wl[0], Wl[0]]).reshape(1, 2 * H)

    dinvp = _sc_deg(edge4, ones16, zeros16)
    xp = x.reshape(N // 2, 2 * D)
    g1p = _tc1(xp, m1, dinvp)
    aggp1 = _sc_agg(g1p.reshape(N, H), edge4, zerosh)
    g2p = _tc2(aggp1.reshape(NC, NP // 2, 2 * H), g1p, dinvp, b1p, m2)
    aggp2 = _sc_agg(g2p.reshape(N, H), edge4, zerosh)
    out = _tc3(aggp2.reshape(NC, NP // 2, 2 * H), g2p, dinvp, b2p, wlp,
               bl.reshape(1, 1))
    return out.reshape(N)


# speedup vs baseline: 3.6833x; 1.0120x over previous
import jax
import jax.numpy as jnp
from jax import lax
from jax.experimental import pallas as pl
from jax.experimental.pallas import tpu as pltpu
from jax.experimental.pallas import tpu_sc as plsc

N = 10000
E = 320000
D = 128
H = 64

NC = 2
NS = 16
NW = NC * NS
CH = 40
NCHUNK = 250
NP = 10240
RPT = NP // NS
KD = 25

_sc_mesh = plsc.VectorSubcoreMesh(
    core_axis_name="c", subcore_axis_name="s", num_cores=NC, num_subcores=NS)


def _sc_deg_body(edge_hbm, ones_hbm, zeros_hbm, out_hbm,
                 dstv, onesv, dbuf, obuf, acc, sem):
    c = lax.axis_index("c")
    s = lax.axis_index("s")
    pltpu.sync_copy(ones_hbm, onesv)
    pltpu.sync_copy(zeros_hbm.at[pl.ds(s * RPT, RPT)],
                    acc.at[pl.ds(s * RPT, RPT)])
    plsc.subcore_barrier()

    for c2 in range(NC):
        pltpu.sync_copy(edge_hbm.at[1, c2, s], dstv)

        def group(i, carry):
            jj = i * KD
            descs = [
                pltpu.async_copy(onesv, acc.at[dstv.at[jj + b]], sem,
                                 add=True)
                for b in range(KD)
            ]
            for d in descs:
                d.wait()
            return carry

        lax.fori_loop(0, NCHUNK // KD, group, 0)
    plsc.subcore_barrier()

    pltpu.sync_copy(acc.at[pl.ds(s * RPT, RPT)], dbuf)

    def post(r, carry):
        d = dbuf[r, :] + 1.0
        i = plsc.bitcast(d, jnp.int32)
        y = plsc.bitcast(0x5F3759DF - (i >> 1), jnp.float32)
        for _ in range(3):
            y = y * (1.5 - 0.5 * d * y * y)
        half = (r % 2) * H
        for k in range(4):
            obuf[r // 2, pl.ds(half + k * 16, 16)] = y
        return carry

    lax.fori_loop(0, RPT, post, 0)

    @pl.when(c == 0)
    def _():
        pltpu.sync_copy(obuf, out_hbm.at[pl.ds(s * (RPT // 2), RPT // 2)])


_sc_deg = pl.kernel(
    _sc_deg_body,
    out_type=jax.ShapeDtypeStruct((NP // 2, 2 * H), jnp.float32),
    mesh=_sc_mesh,
    compiler_params=pltpu.CompilerParams(use_tc_tiling_on_sc=False,
                                         needs_layout_passes=False),
    scratch_types=[
        pltpu.VMEM((NCHUNK, CH), jnp.int32),
        pltpu.VMEM((CH, 16), jnp.float32),
        pltpu.VMEM((RPT, 16), jnp.float32),
        pltpu.VMEM((RPT // 2, 2 * H), jnp.float32),
        pltpu.VMEM_SHARED((NP, 16), jnp.float32),
        pltpu.SemaphoreType.DMA,
    ],
)


def _sc_agg_body(g_hbm, edge_hbm, zeros_hbm, out_hbm,
                 srcv, dstv, rows, acc, gsem, ssem):
    c = lax.axis_index("c")
    s = lax.axis_index("s")
    pltpu.sync_copy(edge_hbm.at[0, c, s], srcv)
    pltpu.sync_copy(edge_hbm.at[1, c, s], dstv)
    pltpu.sync_copy(zeros_hbm.at[pl.ds(s * RPT, RPT)],
                    acc.at[pl.ds(s * RPT, RPT)])
    plsc.subcore_barrier()

    def group(i, carry):
        jj = i * KD
        gd = [
            pltpu.async_copy(g_hbm.at[srcv.at[jj + b]], rows.at[b], gsem)
            for b in range(KD)
        ]
        sd = []
        for b in range(KD):
            gd[b].wait()
            sd.append(pltpu.async_copy(rows.at[b], acc.at[dstv.at[jj + b]],
                                       ssem, add=True))
        for d in sd:
            d.wait()
        return carry

    lax.fori_loop(0, NCHUNK // KD, group, 0)
    plsc.subcore_barrier()
    pltpu.sync_copy(acc.at[pl.ds(s * RPT, RPT)],
                    out_hbm.at[c, pl.ds(s * RPT, RPT)])


_sc_agg = pl.kernel(
    _sc_agg_body,
    out_type=jax.ShapeDtypeStruct((NC, NP, H), jnp.float32),
    mesh=_sc_mesh,
    compiler_params=pltpu.CompilerParams(use_tc_tiling_on_sc=False),
    scratch_types=[
        pltpu.VMEM((NCHUNK, CH), jnp.int32),
        pltpu.VMEM((NCHUNK, CH), jnp.int32),
        pltpu.VMEM((KD, CH, H), jnp.float32),
        pltpu.VMEM_SHARED((NP, H), jnp.float32),
        pltpu.SemaphoreType.DMA,
        pltpu.SemaphoreType.DMA,
    ],
)


BLKP = 5000


def _tc1_body(xp_ref, m1_ref, dinv_ref, g1_ref):
    h = lax.dot_general(xp_ref[...], m1_ref[...], (((1,), (0,)), ((), ())),
                        preferred_element_type=jnp.float32)
    g1_ref[...] = h * dinv_ref[...]


def _tc1(xp, m1, dinv):
    return pl.pallas_call(
        _tc1_body,
        grid=(N // 2 // BLKP,),
        in_specs=[
            pl.BlockSpec((BLKP, 2 * D), lambda i: (i, 0)),
            pl.BlockSpec((2 * D, 2 * H), lambda i: (0, 0)),
            pl.BlockSpec((BLKP, 2 * H), lambda i: (i, 0)),
        ],
        out_specs=pl.BlockSpec((BLKP, 2 * H), lambda i: (i, 0)),
        out_shape=jax.ShapeDtypeStruct((N // 2, 2 * H), jnp.float32),
    )(xp, m1, dinv)


def _tc2_body(agg_ref, g1_ref, dinv_ref, b1_ref, m2_ref, g2_ref):
    a = agg_ref[0] + agg_ref[1] + g1_ref[...]
    h = jnp.maximum(a * dinv_ref[...] + b1_ref[...], 0.0)
    g2 = lax.dot_general(h, m2_ref[...], (((1,), (0,)), ((), ())),
                         preferred_element_type=jnp.float32)
    g2_ref[...] = g2 * dinv_ref[...]


def _tc2(aggv, g1p, dinv, b1p, m2):
    return pl.pallas_call(
        _tc2_body,
        grid=(N // 2 // BLKP,),
        in_specs=[
            pl.BlockSpec((NC, BLKP, 2 * H), lambda i: (0, i, 0)),
            pl.BlockSpec((BLKP, 2 * H), lambda i: (i, 0)),
            pl.BlockSpec((BLKP, 2 * H), lambda i: (i, 0)),
            pl.BlockSpec((1, 2 * H), lambda i: (0, 0)),
            pl.BlockSpec((2 * H, 2 * H), lambda i: (0, 0)),
        ],
        out_specs=pl.BlockSpec((BLKP, 2 * H), lambda i: (i, 0)),
        out_shape=jax.ShapeDtypeStruct((N // 2, 2 * H), jnp.float32),
    )(aggv, g1p, dinv, b1p, m2)


def _tc3_body(agg_ref, g2_ref, dinv_ref, b2_ref, wl_ref, bl_ref, out_ref):
    a = agg_ref[0] + agg_ref[1] + g2_ref[...]
    h = jnp.maximum(a * dinv_ref[...] + b2_ref[...], 0.0)
    t = h * wl_ref[...]
    o0 = jnp.sum(t[:, :H], axis=1, keepdims=True)
    o1 = jnp.sum(t[:, H:], axis=1, keepdims=True)
    out_ref[...] = jnp.concatenate([o0, o1], axis=1) + bl_ref[...]


def _tc3(aggv, g2p, dinv, b2p, wlp, bl):
    return pl.pallas_call(
        _tc3_body,
        grid=(N // 2 // BLKP,),
        in_specs=[
            pl.BlockSpec((NC, BLKP, 2 * H), lambda i: (0, i, 0)),
            pl.BlockSpec((BLKP, 2 * H), lambda i: (i, 0)),
            pl.BlockSpec((BLKP, 2 * H), lambda i: (i, 0)),
            pl.BlockSpec((1, 2 * H), lambda i: (0, 0)),
            pl.BlockSpec((1, 2 * H), lambda i: (0, 0)),
            pl.BlockSpec((1, 1), lambda i: (0, 0)),
        ],
        out_specs=pl.BlockSpec((BLKP, 2), lambda i: (i, 0)),
        out_shape=jax.ShapeDtypeStruct((N // 2, 2), jnp.float32),
    )(aggv, g2p, dinv, b2p, wlp, bl)


def kernel(x, edge_index, W1, b1, W2, b2, Wl, bl):
    f32 = jnp.float32
    edge4 = edge_index.reshape(2, NC, NS, NCHUNK, CH)
    ones16 = jnp.ones((CH, 16), f32)
    zeros16 = jnp.zeros((NP, 16), f32)
    zerosh = jnp.zeros((NP, H), f32)
    m1 = jnp.zeros((2 * D, 2 * H), f32)
    m1 = m1.at[:D, :H].set(W1.T).at[D:, H:].set(W1.T)
    m2 = jnp.zeros((2 * H, 2 * H), f32)
    m2 = m2.at[:H, :H].set(W2.T).at[H:, H:].set(W2.T)
    b1p = jnp.concatenate([b1, b1]).reshape(1, 2 * H)
    b2p = jnp.concatenate([b2, b2]).reshape(1, 2 * H)
    wlp = jnp.concatenate([Wl[0], Wl[0]]).reshape(1, 2 * H)

    dinvp = _sc_deg(edge4, ones16, zeros16)
    xp = x.reshape(N // 2, 2 * D)
    g1p = _tc1(xp, m1, dinvp)
    aggp1 = _sc_agg(g1p.reshape(N, H), edge4, zerosh)
    g2p = _tc2(aggp1.reshape(NC, NP // 2, 2 * H), g1p, dinvp, b1p, m2)
    aggp2 = _sc_agg(g2p.reshape(N, H), edge4, zerosh)
    out = _tc3(aggp2.reshape(NC, NP // 2, 2 * H), g2p, dinvp, b2p, wlp,
               bl.reshape(1, 1))
    return out.reshape(N)


# speedup vs baseline: 3.7178x; 1.0094x over previous
import jax
import jax.numpy as jnp
from jax import lax
from jax.experimental import pallas as pl
from jax.experimental.pallas import tpu as pltpu
from jax.experimental.pallas import tpu_sc as plsc

N = 10000
E = 320000
D = 128
H = 64

NC = 2
NS = 16
NW = NC * NS
CH = 40
NCHUNK = 250
NP = 10240
RPT = NP // NS
KD = 25

_sc_mesh = plsc.VectorSubcoreMesh(
    core_axis_name="c", subcore_axis_name="s", num_cores=NC, num_subcores=NS)


def _sc_deg_body(edge_hbm, ones_hbm, zeros_hbm, out_hbm,
                 dstv, onesv, dbuf, obuf, acc, sem):
    c = lax.axis_index("c")
    s = lax.axis_index("s")
    pltpu.sync_copy(ones_hbm, onesv)
    pltpu.sync_copy(zeros_hbm.at[pl.ds(s * RPT, RPT)],
                    acc.at[pl.ds(s * RPT, RPT)])
    plsc.subcore_barrier()

    for c2 in range(NC):
        pltpu.sync_copy(edge_hbm.at[1, c2, s], dstv)

        def group(i, carry):
            jj = i * KD
            descs = [
                pltpu.async_copy(onesv, acc.at[dstv.at[jj + b]], sem,
                                 add=True)
                for b in range(KD)
            ]
            for d in descs:
                d.wait()
            return carry

        lax.fori_loop(0, NCHUNK // KD, group, 0)
    plsc.subcore_barrier()

    pltpu.sync_copy(acc.at[pl.ds(s * RPT, RPT)], dbuf)

    def post(r, carry):
        d = dbuf[r, :] + 1.0
        i = plsc.bitcast(d, jnp.int32)
        y = plsc.bitcast(0x5F3759DF - (i >> 1), jnp.float32)
        for _ in range(3):
            y = y * (1.5 - 0.5 * d * y * y)
        half = (r % 2) * H
        for k in range(4):
            obuf[r // 2, pl.ds(half + k * 16, 16)] = y
        return carry

    lax.fori_loop(0, RPT, post, 0)

    @pl.when(c == 0)
    def _():
        pltpu.sync_copy(obuf, out_hbm.at[pl.ds(s * (RPT // 2), RPT // 2)])


_sc_deg = pl.kernel(
    _sc_deg_body,
    out_type=jax.ShapeDtypeStruct((NP // 2, 2 * H), jnp.float32),
    mesh=_sc_mesh,
    compiler_params=pltpu.CompilerParams(use_tc_tiling_on_sc=False,
                                         needs_layout_passes=False),
    scratch_types=[
        pltpu.VMEM((NCHUNK, CH), jnp.int32),
        pltpu.VMEM((CH, 16), jnp.float32),
        pltpu.VMEM((RPT, 16), jnp.float32),
        pltpu.VMEM((RPT // 2, 2 * H), jnp.float32),
        pltpu.VMEM_SHARED((NP, 16), jnp.float32),
        pltpu.SemaphoreType.DMA,
    ],
)


def _sc_agg_body(g_hbm, edge_hbm, zeros_hbm, out_hbm,
                 srcv, dstv, rows, acc, gsem, ssem):
    c = lax.axis_index("c")
    s = lax.axis_index("s")
    pltpu.sync_copy(edge_hbm.at[0, c, s], srcv)
    pltpu.sync_copy(edge_hbm.at[1, c, s], dstv)
    pltpu.sync_copy(zeros_hbm.at[pl.ds(s * RPT, RPT)],
                    acc.at[pl.ds(s * RPT, RPT)])
    plsc.subcore_barrier()

    def group(i, carry):
        jj = i * KD
        gd = [
            pltpu.async_copy(g_hbm.at[srcv.at[jj + b]], rows.at[b], gsem)
            for b in range(KD)
        ]
        sd = []
        for b in range(KD):
            gd[b].wait()
            sd.append(pltpu.async_copy(rows.at[b], acc.at[dstv.at[jj + b]],
                                       ssem, add=True))
        for d in sd:
            d.wait()
        return carry

    lax.fori_loop(0, NCHUNK // KD, group, 0)
    plsc.subcore_barrier()
    pltpu.sync_copy(acc.at[pl.ds(s * RPT, RPT)],
                    out_hbm.at[c, pl.ds(s * RPT, RPT)])


_sc_agg = pl.kernel(
    _sc_agg_body,
    out_type=jax.ShapeDtypeStruct((NC, NP, H), jnp.float32),
    mesh=_sc_mesh,
    compiler_params=pltpu.CompilerParams(use_tc_tiling_on_sc=False),
    scratch_types=[
        pltpu.VMEM((NCHUNK, CH), jnp.int32),
        pltpu.VMEM((NCHUNK, CH), jnp.int32),
        pltpu.VMEM((KD, CH, H), jnp.float32),
        pltpu.VMEM_SHARED((NP, H), jnp.float32),
        pltpu.SemaphoreType.DMA,
        pltpu.SemaphoreType.DMA,
    ],
)


BLKP = 5000


def _tc1a_body(xp_ref, m1_ref, h1_ref):
    h1_ref[...] = lax.dot_general(xp_ref[...], m1_ref[...],
                                  (((1,), (0,)), ((), ())),
                                  preferred_element_type=jnp.float32)


def _tc1a(xp, m1):
    return pl.pallas_call(
        _tc1a_body,
        grid=(N // 2 // BLKP,),
        in_specs=[
            pl.BlockSpec((BLKP, 2 * D), lambda i: (i, 0)),
            pl.BlockSpec((2 * D, 2 * H), lambda i: (0, 0)),
        ],
        out_specs=pl.BlockSpec((BLKP, 2 * H), lambda i: (i, 0)),
        out_shape=jax.ShapeDtypeStruct((N // 2, 2 * H), jnp.float32),
    )(xp, m1)


def _tc1b_body(h1_ref, dinv_ref, g1_ref):
    g1_ref[...] = h1_ref[...] * dinv_ref[...]


def _tc1b(h1p, dinv):
    return pl.pallas_call(
        _tc1b_body,
        grid=(N // 2 // BLKP,),
        in_specs=[
            pl.BlockSpec((BLKP, 2 * H), lambda i: (i, 0)),
            pl.BlockSpec((BLKP, 2 * H), lambda i: (i, 0)),
        ],
        out_specs=pl.BlockSpec((BLKP, 2 * H), lambda i: (i, 0)),
        out_shape=jax.ShapeDtypeStruct((N // 2, 2 * H), jnp.float32),
    )(h1p, dinv)


def _tc2_body(agg_ref, g1_ref, dinv_ref, b1_ref, m2_ref, g2_ref):
    a = agg_ref[0] + agg_ref[1] + g1_ref[...]
    h = jnp.maximum(a * dinv_ref[...] + b1_ref[...], 0.0)
    g2 = lax.dot_general(h, m2_ref[...], (((1,), (0,)), ((), ())),
                         preferred_element_type=jnp.float32)
    g2_ref[...] = g2 * dinv_ref[...]


def _tc2(aggv, g1p, dinv, b1p, m2):
    return pl.pallas_call(
        _tc2_body,
        grid=(N // 2 // BLKP,),
        in_specs=[
            pl.BlockSpec((NC, BLKP, 2 * H), lambda i: (0, i, 0)),
            pl.BlockSpec((BLKP, 2 * H), lambda i: (i, 0)),
            pl.BlockSpec((BLKP, 2 * H), lambda i: (i, 0)),
            pl.BlockSpec((1, 2 * H), lambda i: (0, 0)),
            pl.BlockSpec((2 * H, 2 * H), lambda i: (0, 0)),
        ],
        out_specs=pl.BlockSpec((BLKP, 2 * H), lambda i: (i, 0)),
        out_shape=jax.ShapeDtypeStruct((N // 2, 2 * H), jnp.float32),
    )(aggv, g1p, dinv, b1p, m2)


def _tc3_body(agg_ref, g2_ref, dinv_ref, b2_ref, wl_ref, bl_ref, out_ref):
    a = agg_ref[0] + agg_ref[1] + g2_ref[...]
    h = jnp.maximum(a * dinv_ref[...] + b2_ref[...], 0.0)
    t = h * wl_ref[...]
    o0 = jnp.sum(t[:, :H], axis=1, keepdims=True)
    o1 = jnp.sum(t[:, H:], axis=1, keepdims=True)
    out_ref[...] = jnp.concatenate([o0, o1], axis=1) + bl_ref[...]


def _tc3(aggv, g2p, dinv, b2p, wlp, bl):
    return pl.pallas_call(
        _tc3_body,
        grid=(N // 2 // BLKP,),
        in_specs=[
            pl.BlockSpec((NC, BLKP, 2 * H), lambda i: (0, i, 0)),
            pl.BlockSpec((BLKP, 2 * H), lambda i: (i, 0)),
            pl.BlockSpec((BLKP, 2 * H), lambda i: (i, 0)),
            pl.BlockSpec((1, 2 * H), lambda i: (0, 0)),
            pl.BlockSpec((1, 2 * H), lambda i: (0, 0)),
            pl.BlockSpec((1, 1), lambda i: (0, 0)),
        ],
        out_specs=pl.BlockSpec((BLKP, 2), lambda i: (i, 0)),
        out_shape=jax.ShapeDtypeStruct((N // 2, 2), jnp.float32),
    )(aggv, g2p, dinv, b2p, wlp, bl)


def kernel(x, edge_index, W1, b1, W2, b2, Wl, bl):
    f32 = jnp.float32
    edge4 = edge_index.reshape(2, NC, NS, NCHUNK, CH)
    ones16 = jnp.ones((CH, 16), f32)
    zeros16 = jnp.zeros((NP, 16), f32)
    zerosh = jnp.zeros((NP, H), f32)
    m1 = jnp.zeros((2 * D, 2 * H), f32)
    m1 = m1.at[:D, :H].set(W1.T).at[D:, H:].set(W1.T)
    m2 = jnp.zeros((2 * H, 2 * H), f32)
    m2 = m2.at[:H, :H].set(W2.T).at[H:, H:].set(W2.T)
    b1p = jnp.concatenate([b1, b1]).reshape(1, 2 * H)
    b2p = jnp.concatenate([b2, b2]).reshape(1, 2 * H)
    wlp = jnp.concatenate([Wl[0], Wl[0]]).reshape(1, 2 * H)

    dinvp = _sc_deg(edge4, ones16, zeros16)
    xp = x.reshape(N // 2, 2 * D)
    h1p = _tc1a(xp, m1)
    g1p = _tc1b(h1p, dinvp)
    aggp1 = _sc_agg(g1p.reshape(N, H), edge4, zerosh)
    g2p = _tc2(aggp1.reshape(NC, NP // 2, 2 * H), g1p, dinvp, b1p, m2)
    aggp2 = _sc_agg(g2p.reshape(N, H), edge4, zerosh)
    out = _tc3(aggp2.reshape(NC, NP // 2, 2 * H), g2p, dinvp, b2p, wlp,
               bl.reshape(1, 1))
    return out.reshape(N)
